# baseline (speedup 1.0000x reference)
import functools

import jax
import jax.numpy as jnp
from jax import lax
from jax.experimental import pallas as pl
from jax.experimental.pallas import tpu as pltpu
from jax.experimental.pallas import tpu_sc as plsc

_LANES = 16
_IDX_CHUNK = 128


@functools.lru_cache(maxsize=None)
def _make_depad_kernel(batch, list_len):
    info = plsc.get_sparse_core_info()
    num_workers = info.num_cores * info.num_subcores
    rows_per_w = batch // num_workers
    mesh = plsc.VectorSubcoreMesh(core_axis_name="c", subcore_axis_name="s")

    tail = list_len - _IDX_CHUNK

    @functools.partial(
        pl.kernel,
        out_type=jax.ShapeDtypeStruct((batch * list_len,), jnp.int32),
        mesh=mesh,
        scratch_types=[
            pltpu.VMEM((rows_per_w, _IDX_CHUNK), jnp.int32),
            pltpu.VMEM((rows_per_w, tail), jnp.int32),
            pltpu.SemaphoreType.DMA,
            pltpu.SemaphoreType.DMA,
        ],
    )
    def depad(iid_hbm, out_hbm, b0, b1, sem_in, sem_out):
        wid = lax.axis_index("s") * info.num_cores + lax.axis_index("c")
        rbase = wid * rows_per_w
        pltpu.async_copy(
            iid_hbm.at[pl.ds(rbase, rows_per_w), pl.ds(0, _IDX_CHUNK)],
            b0, sem_in)
        pltpu.async_copy(
            iid_hbm.at[pl.ds(rbase, rows_per_w), pl.ds(_IDX_CHUNK, tail)],
            b1, sem_in).wait()
        pltpu.make_async_copy(
            iid_hbm.at[pl.ds(rbase, rows_per_w), pl.ds(0, _IDX_CHUNK)],
            b0, sem_in).wait()

        def row(r, carry):
            pltpu.async_copy(
                b0.at[r],
                out_hbm.at[pl.ds((rbase + r) * list_len, _IDX_CHUNK)],
                sem_out)
            pltpu.async_copy(
                b1.at[r],
                out_hbm.at[pl.ds((rbase + r) * list_len + _IDX_CHUNK, tail)],
                sem_out)
            return carry

        lax.fori_loop(0, rows_per_w, row, 0)

        def drain(r, carry):
            pltpu.make_async_copy(
                b0.at[r],
                out_hbm.at[pl.ds((rbase + r) * list_len, _IDX_CHUNK)],
                sem_out).wait()
            pltpu.make_async_copy(
                b1.at[r],
                out_hbm.at[pl.ds((rbase + r) * list_len + _IDX_CHUNK, tail)],
                sem_out).wait()
            return carry

        lax.fori_loop(0, rows_per_w, drain, 0)

    return depad


@functools.lru_cache(maxsize=None)
def _make_repad_kernel(batch, list_len):
    info = plsc.get_sparse_core_info()
    num_workers = info.num_cores * info.num_subcores
    rows_per_w = batch // num_workers
    mesh = plsc.VectorSubcoreMesh(core_axis_name="c", subcore_axis_name="s")

    tail = list_len - _IDX_CHUNK
    n_tiles = rows_per_w // 8

    @functools.partial(
        pl.kernel,
        out_type=jax.ShapeDtypeStruct((batch, list_len), jnp.float32),
        mesh=mesh,
        scratch_types=[
            pltpu.VMEM((rows_per_w, _IDX_CHUNK), jnp.float32),
            pltpu.VMEM((rows_per_w, tail), jnp.float32),
            pltpu.SemaphoreType.DMA,
            pltpu.SemaphoreType.DMA,
        ],
    )
    def repad(flat_hbm, out_hbm, b0, b1, sem_in, sem_out):
        wid = lax.axis_index("s") * info.num_cores + lax.axis_index("c")
        rbase = wid * rows_per_w
        pltpu.async_copy(
            flat_hbm.at[pl.ds(rbase, rows_per_w), pl.ds(0, _IDX_CHUNK)],
            b0, sem_in)
        pltpu.async_copy(
            flat_hbm.at[pl.ds(rbase, rows_per_w), pl.ds(_IDX_CHUNK, tail)],
            b1, sem_in).wait()
        pltpu.make_async_copy(
            flat_hbm.at[pl.ds(rbase, rows_per_w), pl.ds(0, _IDX_CHUNK)],
            b0, sem_in).wait()

        def tile(t, carry):
            r8 = t * 8
            pltpu.async_copy(
                b0.at[pl.ds(r8, 8), :],
                out_hbm.at[pl.ds(rbase + r8, 8), pl.ds(0, _IDX_CHUNK)],
                sem_out)
            pltpu.async_copy(
                b1.at[pl.ds(r8, 8), :],
                out_hbm.at[pl.ds(rbase + r8, 8), pl.ds(_IDX_CHUNK, tail)],
                sem_out)
            return carry

        lax.fori_loop(0, n_tiles, tile, 0)

        def drain(t, carry):
            r8 = t * 8
            pltpu.make_async_copy(
                b0.at[pl.ds(r8, 8), :],
                out_hbm.at[pl.ds(rbase + r8, 8), pl.ds(0, _IDX_CHUNK)],
                sem_out).wait()
            pltpu.make_async_copy(
                b1.at[pl.ds(r8, 8), :],
                out_hbm.at[pl.ds(rbase + r8, 8), pl.ds(_IDX_CHUNK, tail)],
                sem_out).wait()
            return carry

        lax.fori_loop(0, n_tiles, drain, 0)

    return repad


@functools.lru_cache(maxsize=None)
def _make_sc_kernel(batch, list_len, dim):
    info = plsc.get_sparse_core_info()
    num_workers = info.num_cores * info.num_subcores
    users_per_w = batch // num_workers
    assert batch % num_workers == 0
    ngroups = (list_len + _LANES - 1) // _LANES
    pad_rows = ngroups * _LANES
    sub_sizes = []
    rem = list_len
    while rem > 0:
        s = min(_IDX_CHUNK, rem)
        sub_sizes.append(s)
        rem -= s

    mesh = plsc.VectorSubcoreMesh(core_axis_name="c", subcore_axis_name="s")

    @functools.partial(
        pl.kernel,
        out_type=jax.ShapeDtypeStruct((batch, list_len), jnp.float32),
        mesh=mesh,
        compiler_params=pltpu.CompilerParams(
            needs_layout_passes=False, use_tc_tiling_on_sc=False),
        scratch_types=[
            pltpu.VMEM((users_per_w,), jnp.int32),
            pltpu.VMEM((users_per_w, dim), jnp.float32),
            pltpu.VMEM((users_per_w * list_len,), jnp.int32),
            pltpu.VMEM((pad_rows, dim), jnp.float32),
            pltpu.VMEM((pad_rows, dim), jnp.float32),
            pltpu.VMEM((pad_rows,), jnp.float32),
            pltpu.SemaphoreType.DMA,
            pltpu.SemaphoreType.DMA,
            pltpu.SemaphoreType.DMA,
        ],
    )
    def sc_kernel(uid_hbm, iid_hbm, utab_hbm, itab_hbm, out_hbm,
                  uidx, urows, iidx, rows0, rows1, scores, sem0, sem1, semu):
        rows = (rows0, rows1)
        sems = (sem0, sem1)
        wid = lax.axis_index("s") * info.num_cores + lax.axis_index("c")
        ubase = wid * users_per_w

        pltpu.sync_copy(uid_hbm.at[pl.ds(ubase, users_per_w)], uidx)
        pltpu.sync_copy(
            iid_hbm.at[pl.ds(ubase * list_len, users_per_w * list_len)], iidx)
        pltpu.async_copy(utab_hbm.at[uidx], urows, semu).wait()

        def gather_descs(u, b):
            off = pl.multiple_of(u * list_len, 8)
            descs = []
            pos = 0
            for s in sub_sizes:
                descs.append(pltpu.make_async_copy(
                    itab_hbm.at[iidx.at[pl.ds(off + pos, s)]],
                    rows[b].at[pl.ds(pos, s)],
                    sems[b]))
                pos += s
            return descs

        def start_gather(u, b):
            for d in gather_descs(u, b):
                d.start()

        def wait_gather(u, b):
            for d in gather_descs(u, b):
                d.wait()

        def compute(u, rowsb):
            uvecs = [urows[u, pl.ds(h * _LANES, _LANES)]
                     for h in range(dim // _LANES)]
            uscal = [uvecs[d // _LANES][d % _LANES] for d in range(dim)]
            for g in range(ngroups):
                ridx = lax.iota(jnp.int32, _LANES) + (g * _LANES)
                acc = jnp.zeros((_LANES,), jnp.float32)
                for d in range(dim):
                    col = jnp.full((_LANES,), d, jnp.int32)
                    v = plsc.load_gather(rowsb, [ridx, col])
                    acc = acc + v * uscal[d]
                scores[pl.ds(g * _LANES, _LANES)] = acc

        start_gather(0, 0)

        def body(uu, carry):
            for b in range(2):
                u = uu * 2 + b
                wait_gather(u, b)

                @pl.when(u + 1 < users_per_w)
                def _prefetch():
                    start_gather(u + 1, 1 - b)

                compute(u, rows[b])
                pltpu.sync_copy(
                    scores.at[pl.ds(0, list_len)],
                    out_hbm.at[ubase + u])
            return carry

        lax.fori_loop(0, users_per_w // 2, body, 0)

    return sc_kernel


def kernel(user_id, item_ids, user_table, item_table):
    batch, list_len = item_ids.shape
    dim = user_table.shape[1]
    ids_flat = _make_depad_kernel(batch, list_len)(item_ids)
    scores_flat = _make_sc_kernel(batch, list_len, dim)(
        user_id, ids_flat, user_table, item_table)
    return _make_repad_kernel(batch, list_len)(scores_flat)
